# baseline (speedup 1.0000x reference)
import functools

import jax
import jax.numpy as jnp
from jax import lax
from jax.experimental import pallas as pl
from jax.experimental.pallas import tpu as pltpu
from jax.experimental.pallas import tpu_sc as plsc

NN = 10000
EE = 320000
DD = 128
GG = 64
NC = 2
NS = 16
CHUNK = 128
CPT = 80
EP = NC * NS * CPT * CHUNK
ROWS_PAD = 10112
RPT = ROWS_PAD // NS
RB = 1000
NB = NN // RB


def _sc_mesh():
    return plsc.VectorSubcoreMesh(
        core_axis_name="c", subcore_axis_name="s",
        num_cores=NC, num_subcores=NS)


def _load_chunk_idx(big_ref, j, idx1):
    for kk in range(8):
        idx1[pl.ds(kk * 16, 16)] = big_ref[j, pl.ds(kk * 16, 16)]


def _fill_rows(rows_v, val):
    def fill(i, _):
        for kk in range(8):
            rows_v[i, pl.ds(kk * 16, 16)] = jnp.full((16,), val, jnp.float32)
        return 0
    lax.fori_loop(0, CHUNK, fill, 0)


def _zero_acc_slice(rows_v, acc, s):
    for q in range(4):
        pltpu.sync_copy(rows_v, acc.at[pl.ds(s * RPT + q * CHUNK, CHUNK)])
    pltpu.sync_copy(rows_v.at[pl.ds(0, RPT - 4 * CHUNK)],
                    acc.at[pl.ds(s * RPT + 4 * CHUNK, RPT - 4 * CHUNK)])


def _deg_body(dst_hbm, out_hbm, dst_v, idx1, rows_v, dacc):
    c = lax.axis_index("c")
    s = lax.axis_index("s")

    _fill_rows(rows_v, 0.0)
    pltpu.sync_copy(dst_hbm.at[c, s], dst_v)
    _zero_acc_slice(rows_v, dacc, s)
    plsc.subcore_barrier()
    _fill_rows(rows_v, 1.0)

    def body(j, _):
        _load_chunk_idx(dst_v, j, idx1)
        pltpu.sync_copy(rows_v, dacc.at[idx1], add=True)
        return 0
    lax.fori_loop(0, CPT, body, 0)

    plsc.subcore_barrier()
    pltpu.sync_copy(dacc.at[pl.ds(s * RPT, RPT)],
                    out_hbm.at[c, pl.ds(s * RPT, RPT)])


def _sc_deg(dstI):
    k = pl.kernel(
        _deg_body,
        out_type=jax.ShapeDtypeStruct((NC, ROWS_PAD, DD), jnp.float32),
        mesh=_sc_mesh(),
        scratch_types=[
            pltpu.VMEM((CPT, CHUNK), jnp.int32),
            pltpu.VMEM((CHUNK,), jnp.int32),
            pltpu.VMEM((CHUNK, DD), jnp.float32),
            pltpu.VMEM_SHARED((ROWS_PAD, DD), jnp.float32),
        ],
    )
    return k(dstI)


def _spmm_body(hp_hbm, src_hbm, dst_hbm, out_hbm,
               dst_v, idx_s, idx_d, rows, acc, gsem, isem):
    c = lax.axis_index("c")
    s = lax.axis_index("s")

    _fill_rows(rows[0], 0.0)
    _fill_rows(rows[1], 0.0)
    pltpu.sync_copy(dst_hbm.at[c, s], dst_v)
    _zero_acc_slice(rows[0], acc, s)
    plsc.subcore_barrier()

    def _gather_chunk(q):
        for u in range(4):
            pltpu.async_copy(hp_hbm.at[idx_s[q].at[pl.ds(32 * u, 32)]],
                             rows[q].at[pl.ds(32 * u, 32)], gsem[q][u])

    def _wait_chunk(q):
        for u in range(4):
            pltpu.make_async_copy(hp_hbm.at[idx_s[q].at[pl.ds(32 * u, 32)]],
                                  rows[q].at[pl.ds(32 * u, 32)],
                                  gsem[q][u]).wait()

    pltpu.sync_copy(src_hbm.at[c, s, 0, 0], idx_s[0])
    _gather_chunk(0)
    pltpu.async_copy(src_hbm.at[c, s, 1, 0], idx_s[1], isem[1])

    def body(m, _):
        for q in range(2):
            j = 2 * m + q
            qo = 1 - q
            pltpu.make_async_copy(src_hbm.at[c, s, 0, 0], idx_s[qo],
                                  isem[qo]).wait()
            _gather_chunk(qo)
            _wait_chunk(q)
            pltpu.async_copy(src_hbm.at[c, s, lax.rem(j + 2, CPT), 0],
                             idx_s[q], isem[q])
            _load_chunk_idx(dst_v, j, idx_d)
            pltpu.sync_copy(rows[q], acc.at[idx_d], add=True)
        return 0
    lax.fori_loop(0, CPT // 2, body, 0)

    _wait_chunk(0)
    pltpu.make_async_copy(src_hbm.at[c, s, 0, 0], idx_s[1], isem[1]).wait()

    plsc.subcore_barrier()
    pltpu.sync_copy(acc.at[pl.ds(s * RPT, RPT)],
                    out_hbm.at[c, pl.ds(s * RPT, RPT)])


def _sc_spmm(hp, srcI, dstI):
    k = pl.kernel(
        _spmm_body,
        out_type=jax.ShapeDtypeStruct((NC, ROWS_PAD, DD), jnp.float32),
        mesh=_sc_mesh(),
        scratch_types=[
            pltpu.VMEM((CPT, CHUNK), jnp.int32),
            tuple(pltpu.VMEM((CHUNK,), jnp.int32) for _ in range(2)),
            pltpu.VMEM((CHUNK,), jnp.int32),
            tuple(pltpu.VMEM((CHUNK, DD), jnp.float32) for _ in range(2)),
            pltpu.VMEM_SHARED((ROWS_PAD, DD), jnp.float32),
            tuple(tuple(pltpu.SemaphoreType.DMA for _ in range(4))
                  for _ in range(2)),
            tuple(pltpu.SemaphoreType.DMA for _ in range(2)),
        ],
    )
    return k(hp, srcI, dstI)


def _dinv_of(degp_blk):
    dsum = degp_blk[0][:, 0:1] + degp_blk[1][:, 0:1] + 1.0
    return lax.rsqrt(dsum)


def _prologue_body(x_ref, degp_ref, w_ref, hp_ref):
    dinv = _dinv_of(degp_ref)
    hp_ref[...] = dinv * jnp.dot(x_ref[...], w_ref[...],
                                 preferred_element_type=jnp.float32)


def _tc_prologue(x, degp, W0):
    return pl.pallas_call(
        _prologue_body,
        grid=(NB,),
        in_specs=[
            pl.BlockSpec((RB, DD), lambda i: (i, 0)),
            pl.BlockSpec((NC, RB, DD), lambda i: (0, i, 0)),
            pl.BlockSpec((DD, DD), lambda i: (0, 0)),
        ],
        out_specs=pl.BlockSpec((RB, DD), lambda i: (i, 0)),
        out_shape=jax.ShapeDtypeStruct((NN, DD), jnp.float32),
    )(x, degp, W0)


def _layer_block(accp_ref, hp_ref, degp_ref, b_ref, g_ref, bb_ref, resid_ref):
    dinv = _dinv_of(degp_ref)
    conv = dinv * (accp_ref[0] + accp_ref[1] + hp_ref[...]) + b_ref[...]
    mu = jnp.mean(conv, axis=1, keepdims=True)
    cc = conv - mu
    var = jnp.mean(cc * cc, axis=1, keepdims=True)
    t = cc * lax.rsqrt(var + 1e-5) * g_ref[...] + bb_ref[...]
    t = jnp.where(t > 0, t, 0.1 * t)
    if resid_ref is not None:
        t = t + resid_ref[...]
    return t, dinv


def _epi_body(has_resid, *refs):
    if has_resid:
        (accp_ref, hp_ref, degp_ref, b_ref, g_ref, bb_ref, wn_ref,
         resid_ref, out_ref, hpn_ref) = refs
    else:
        (accp_ref, hp_ref, degp_ref, b_ref, g_ref, bb_ref, wn_ref,
         out_ref, hpn_ref) = refs
        resid_ref = None
    t, dinv = _layer_block(accp_ref, hp_ref, degp_ref, b_ref, g_ref,
                           bb_ref, resid_ref)
    out_ref[...] = t
    hpn_ref[...] = dinv * jnp.dot(t, wn_ref[...],
                                  preferred_element_type=jnp.float32)


def _tc_epilogue(accp, hp, degp, b, g, bb, Wn, resid):
    has_resid = resid is not None
    row = pl.BlockSpec((RB, DD), lambda i: (i, 0))
    vec = pl.BlockSpec((1, DD), lambda i: (0, 0))
    in_specs = [
        pl.BlockSpec((NC, RB, DD), lambda i: (0, i, 0)),
        row,
        pl.BlockSpec((NC, RB, DD), lambda i: (0, i, 0)),
        vec, vec, vec,
        pl.BlockSpec((DD, DD), lambda i: (0, 0)),
    ]
    args = [accp, hp, degp, b, g, bb, Wn]
    if has_resid:
        in_specs.append(row)
        args.append(resid)
    return pl.pallas_call(
        functools.partial(_epi_body, has_resid),
        grid=(NB,),
        in_specs=in_specs,
        out_specs=[row, row],
        out_shape=[jax.ShapeDtypeStruct((NN, DD), jnp.float32),
                   jax.ShapeDtypeStruct((NN, DD), jnp.float32)],
    )(*args)


def _final_body(accp_ref, hp_ref, degp_ref, b_ref, g_ref, bb_ref, resid_ref,
                ba_ref, bl_ref, wlmax_ref, wlmean_ref, blin_ref, out_ref,
                maxacc, sumacc, cntacc):
    i = pl.program_id(0)

    @pl.when(i == 0)
    def _init():
        maxacc[...] = jnp.full((GG, 1, DD), -jnp.inf, jnp.float32)
        sumacc[...] = jnp.zeros((GG, DD), jnp.float32)
        cntacc[...] = jnp.zeros((GG, DD), jnp.float32)

    t, _ = _layer_block(accp_ref, hp_ref, degp_ref, b_ref, g_ref, bb_ref,
                        resid_ref)

    bvec = ba_ref[0]
    blane = bl_ref[0]
    onehot_t = (lax.broadcasted_iota(jnp.int32, (GG, RB), 0)
                == blane).astype(jnp.float32)
    sumacc[...] = sumacc[...] + jnp.dot(onehot_t, t,
                                        preferred_element_type=jnp.float32)
    cntacc[...] = cntacc[...] + jnp.sum(onehot_t, axis=1, keepdims=True)

    g_lo = jnp.min(bvec)
    g_hi = jnp.max(bvec)

    def seg(gidx, _):
        m = jnp.max(jnp.where(bvec == gidx, t, -jnp.inf), axis=0,
                    keepdims=True)
        maxacc[gidx] = jnp.maximum(maxacc[gidx], m)
        return 0
    lax.fori_loop(g_lo, g_hi + 1, seg, 0)

    @pl.when(i == NB - 1)
    def _emit():
        meanp = sumacc[...] / jnp.maximum(cntacc[...], 1.0)
        maxp = maxacc[...][:, 0, :]
        out_ref[...] = (
            jnp.dot(maxp, wlmax_ref[...], preferred_element_type=jnp.float32)
            + jnp.dot(meanp, wlmean_ref[...],
                      preferred_element_type=jnp.float32)
            + blin_ref[...])


def _tc_final(accp, hp, degp, b, g, bb, resid, batch_a, batch_b,
              Wl_max, Wl_mean, b_lin):
    row = pl.BlockSpec((RB, DD), lambda i: (i, 0))
    vec = pl.BlockSpec((1, DD), lambda i: (0, 0))
    mat = pl.BlockSpec((DD, DD), lambda i: (0, 0))
    return pl.pallas_call(
        _final_body,
        grid=(NB,),
        in_specs=[
            pl.BlockSpec((NC, RB, DD), lambda i: (0, i, 0)),
            row,
            pl.BlockSpec((NC, RB, DD), lambda i: (0, i, 0)),
            vec, vec, vec,
            row,
            pl.BlockSpec((1, RB, 1), lambda i: (i, 0, 0)),
            pl.BlockSpec((1, 1, RB), lambda i: (i, 0, 0)),
            mat, mat, vec,
        ],
        out_specs=pl.BlockSpec((GG, DD), lambda i: (0, 0)),
        out_shape=jax.ShapeDtypeStruct((GG, DD), jnp.float32),
        scratch_shapes=[
            pltpu.VMEM((GG, 1, DD), jnp.float32),
            pltpu.VMEM((GG, DD), jnp.float32),
            pltpu.VMEM((GG, DD), jnp.float32),
        ],
    )(accp, hp, degp, b, g, bb, resid, batch_a, batch_b,
      Wl_max, Wl_mean, b_lin)


def kernel(x, edge_index, edge_attr, batch,
           W_gcn0, b_gcn0, W_gcn1, b_gcn1, W_gcn2, b_gcn2,
           ln_g0, ln_b0, ln_g1, ln_b1, ln_g2, ln_b2,
           W_lin, b_lin):
    src = edge_index[0].astype(jnp.int32)
    dst = edge_index[1].astype(jnp.int32)
    pad = EP - EE
    srcI = jnp.concatenate(
        [src, jnp.zeros((pad,), jnp.int32)]).reshape(NC, NS, CPT, 1, CHUNK)
    dstI = jnp.concatenate(
        [dst, jnp.full((pad,), NN, jnp.int32)]).reshape(NC, NS, CPT, CHUNK)

    degp = _sc_deg(dstI)
    b0 = b_gcn0.reshape(1, DD)
    b1 = b_gcn1.reshape(1, DD)
    b2 = b_gcn2.reshape(1, DD)
    g0, g1, g2 = (v.reshape(1, DD) for v in (ln_g0, ln_g1, ln_g2))
    bb0, bb1, bb2 = (v.reshape(1, DD) for v in (ln_b0, ln_b1, ln_b2))

    hp0 = _tc_prologue(x, degp, W_gcn0)
    acc0 = _sc_spmm(hp0, srcI, dstI)
    out0, hp1 = _tc_epilogue(acc0, hp0, degp, b0, g0, bb0, W_gcn1, None)
    acc1 = _sc_spmm(hp1, srcI, dstI)
    out1, hp2 = _tc_epilogue(acc1, hp1, degp, b1, g1, bb1, W_gcn2, out0)
    acc2 = _sc_spmm(hp2, srcI, dstI)

    batch_i = batch.astype(jnp.int32)
    batch_a = batch_i.reshape(NB, RB, 1)
    batch_b = batch_i.reshape(NB, 1, RB)
    return _tc_final(acc2, hp2, degp, b2, g2, bb2, out1, batch_a, batch_b,
                     W_lin[:DD], W_lin[DD:], b_lin.reshape(1, DD))


# speedup vs baseline: 1.1561x; 1.1561x over previous
import functools

import jax
import jax.numpy as jnp
from jax import lax
from jax.experimental import pallas as pl
from jax.experimental.pallas import tpu as pltpu
from jax.experimental.pallas import tpu_sc as plsc

NN = 10000
EE = 320000
DD = 128
GG = 64
NC = 2
NS = 16
CHUNK = 128
CPT = 80
EP = NC * NS * CPT * CHUNK
ROWS_PAD = 10112
RPT = ROWS_PAD // NS
RB = 1000
NB = NN // RB


def _sc_mesh():
    return plsc.VectorSubcoreMesh(
        core_axis_name="c", subcore_axis_name="s",
        num_cores=NC, num_subcores=NS)


def _load_chunk_idx(big_ref, j, idx1):
    for kk in range(8):
        idx1[pl.ds(kk * 16, 16)] = big_ref[j, pl.ds(kk * 16, 16)]


def _fill_rows(rows_v, val):
    def fill(i, _):
        for kk in range(8):
            rows_v[i, pl.ds(kk * 16, 16)] = jnp.full((16,), val, jnp.float32)
        return 0
    lax.fori_loop(0, CHUNK, fill, 0)


def _zero_acc_slice(rows_v, acc, s):
    for q in range(4):
        pltpu.sync_copy(rows_v, acc.at[pl.ds(s * RPT + q * CHUNK, CHUNK)])
    pltpu.sync_copy(rows_v.at[pl.ds(0, RPT - 4 * CHUNK)],
                    acc.at[pl.ds(s * RPT + 4 * CHUNK, RPT - 4 * CHUNK)])


def _deg_body(dst_hbm, out_hbm, dst_v, idx1, rows_v, dacc):
    c = lax.axis_index("c")
    s = lax.axis_index("s")

    _fill_rows(rows_v, 0.0)
    pltpu.sync_copy(dst_hbm.at[c, s], dst_v)
    _zero_acc_slice(rows_v, dacc, s)
    plsc.subcore_barrier()
    _fill_rows(rows_v, 1.0)

    def body(j, _):
        _load_chunk_idx(dst_v, j, idx1)
        pltpu.sync_copy(rows_v, dacc.at[idx1], add=True)
        return 0
    lax.fori_loop(0, CPT, body, 0)

    plsc.subcore_barrier()
    pltpu.sync_copy(dacc.at[pl.ds(s * RPT, RPT)],
                    out_hbm.at[c, pl.ds(s * RPT, RPT)])


def _sc_deg(dstI):
    k = pl.kernel(
        _deg_body,
        out_type=jax.ShapeDtypeStruct((NC, ROWS_PAD, DD), jnp.float32),
        mesh=_sc_mesh(),
        scratch_types=[
            pltpu.VMEM((CPT, CHUNK), jnp.int32),
            pltpu.VMEM((CHUNK,), jnp.int32),
            pltpu.VMEM((CHUNK, DD), jnp.float32),
            pltpu.VMEM_SHARED((ROWS_PAD, DD), jnp.float32),
        ],
    )
    return k(dstI)


def _spmm_body(hp_hbm, src_hbm, dst_hbm, out_hbm,
               dst_v, idx_s, idx_d, rows, acc, gsem, isem):
    c = lax.axis_index("c")
    s = lax.axis_index("s")

    _fill_rows(rows[0], 0.0)
    _fill_rows(rows[1], 0.0)
    pltpu.sync_copy(dst_hbm.at[c, s], dst_v)
    _zero_acc_slice(rows[0], acc, s)
    plsc.subcore_barrier()

    def _gather_chunk(q):
        for u in range(4):
            pltpu.async_copy(hp_hbm.at[idx_s[q].at[pl.ds(32 * u, 32)]],
                             rows[q].at[pl.ds(32 * u, 32)], gsem[q][u])

    def _wait_chunk(q):
        for u in range(4):
            pltpu.make_async_copy(hp_hbm.at[idx_s[q].at[pl.ds(32 * u, 32)]],
                                  rows[q].at[pl.ds(32 * u, 32)],
                                  gsem[q][u]).wait()

    pltpu.sync_copy(src_hbm.at[c, s, 0, 0], idx_s[0])
    _gather_chunk(0)
    pltpu.async_copy(src_hbm.at[c, s, 1, 0], idx_s[1], isem[1])

    def body(m, _):
        for q in range(2):
            j = 2 * m + q
            qo = 1 - q
            pltpu.make_async_copy(src_hbm.at[c, s, 0, 0], idx_s[qo],
                                  isem[qo]).wait()
            _gather_chunk(qo)
            _wait_chunk(q)
            pltpu.async_copy(src_hbm.at[c, s, lax.rem(j + 2, CPT), 0],
                             idx_s[q], isem[q])
            _load_chunk_idx(dst_v, j, idx_d)
            pltpu.sync_copy(rows[q], acc.at[idx_d], add=True)
        return 0
    lax.fori_loop(0, CPT // 2, body, 0)

    _wait_chunk(0)
    pltpu.make_async_copy(src_hbm.at[c, s, 0, 0], idx_s[1], isem[1]).wait()

    plsc.subcore_barrier()
    pltpu.sync_copy(acc.at[pl.ds(s * RPT, RPT)],
                    out_hbm.at[c, pl.ds(s * RPT, RPT)])


def _sc_spmm(hp, srcI, dstI):
    k = pl.kernel(
        _spmm_body,
        out_type=jax.ShapeDtypeStruct((NC, ROWS_PAD, DD), jnp.float32),
        mesh=_sc_mesh(),
        scratch_types=[
            pltpu.VMEM((CPT, CHUNK), jnp.int32),
            tuple(pltpu.VMEM((CHUNK,), jnp.int32) for _ in range(2)),
            pltpu.VMEM((CHUNK,), jnp.int32),
            tuple(pltpu.VMEM((CHUNK, DD), jnp.float32) for _ in range(2)),
            pltpu.VMEM_SHARED((ROWS_PAD, DD), jnp.float32),
            tuple(tuple(pltpu.SemaphoreType.DMA for _ in range(4))
                  for _ in range(2)),
            tuple(pltpu.SemaphoreType.DMA for _ in range(2)),
        ],
    )
    return k(hp, srcI, dstI)


def _dinv_of(degp_blk):
    dsum = degp_blk[0][:, 0:1] + degp_blk[1][:, 0:1] + 1.0
    return lax.rsqrt(dsum)


def _prologue_body(x_ref, degp_ref, w_ref, hp_ref):
    dinv = _dinv_of(degp_ref)
    hp_ref[...] = dinv * jnp.dot(x_ref[...], w_ref[...],
                                 preferred_element_type=jnp.float32)


def _tc_prologue(x, degp, W0):
    return pl.pallas_call(
        _prologue_body,
        grid=(NB,),
        in_specs=[
            pl.BlockSpec((RB, DD), lambda i: (i, 0)),
            pl.BlockSpec((NC, RB, DD), lambda i: (0, i, 0)),
            pl.BlockSpec((DD, DD), lambda i: (0, 0)),
        ],
        out_specs=pl.BlockSpec((RB, DD), lambda i: (i, 0)),
        out_shape=jax.ShapeDtypeStruct((NN, DD), jnp.float32),
    )(x, degp, W0)


def _layer_block(accp_ref, hp_ref, degp_ref, b_ref, g_ref, bb_ref, resid_ref):
    dinv = _dinv_of(degp_ref)
    conv = dinv * (accp_ref[0] + accp_ref[1] + hp_ref[...]) + b_ref[...]
    mu = jnp.mean(conv, axis=1, keepdims=True)
    cc = conv - mu
    var = jnp.mean(cc * cc, axis=1, keepdims=True)
    t = cc * lax.rsqrt(var + 1e-5) * g_ref[...] + bb_ref[...]
    t = jnp.where(t > 0, t, 0.1 * t)
    if resid_ref is not None:
        t = t + resid_ref[...]
    return t, dinv


def _epi_body(has_resid, *refs):
    if has_resid:
        (accp_ref, hp_ref, degp_ref, b_ref, g_ref, bb_ref, wn_ref,
         resid_ref, out_ref, hpn_ref) = refs
    else:
        (accp_ref, hp_ref, degp_ref, b_ref, g_ref, bb_ref, wn_ref,
         out_ref, hpn_ref) = refs
        resid_ref = None
    t, dinv = _layer_block(accp_ref, hp_ref, degp_ref, b_ref, g_ref,
                           bb_ref, resid_ref)
    out_ref[...] = t
    hpn_ref[...] = dinv * jnp.dot(t, wn_ref[...],
                                  preferred_element_type=jnp.float32)


def _tc_epilogue(accp, hp, degp, b, g, bb, Wn, resid):
    has_resid = resid is not None
    row = pl.BlockSpec((RB, DD), lambda i: (i, 0))
    vec = pl.BlockSpec((1, DD), lambda i: (0, 0))
    in_specs = [
        pl.BlockSpec((NC, RB, DD), lambda i: (0, i, 0)),
        row,
        pl.BlockSpec((NC, RB, DD), lambda i: (0, i, 0)),
        vec, vec, vec,
        pl.BlockSpec((DD, DD), lambda i: (0, 0)),
    ]
    args = [accp, hp, degp, b, g, bb, Wn]
    aliases = {1: 1}
    if has_resid:
        in_specs.append(row)
        args.append(resid)
        aliases[7] = 0
    return pl.pallas_call(
        functools.partial(_epi_body, has_resid),
        grid=(NB,),
        in_specs=in_specs,
        out_specs=[row, row],
        out_shape=[jax.ShapeDtypeStruct((NN, DD), jnp.float32),
                   jax.ShapeDtypeStruct((NN, DD), jnp.float32)],
        input_output_aliases=aliases,
    )(*args)


def _final_body(accp_ref, hp_ref, degp_ref, b_ref, g_ref, bb_ref, resid_ref,
                ba_ref, bl_ref, wlmax_ref, wlmean_ref, blin_ref, out_ref,
                maxacc, sumacc, cntacc):
    i = pl.program_id(0)

    @pl.when(i == 0)
    def _init():
        maxacc[...] = jnp.full((GG, 1, DD), -jnp.inf, jnp.float32)
        sumacc[...] = jnp.zeros((GG, DD), jnp.float32)
        cntacc[...] = jnp.zeros((GG, DD), jnp.float32)

    t, _ = _layer_block(accp_ref, hp_ref, degp_ref, b_ref, g_ref, bb_ref,
                        resid_ref)

    bvec = ba_ref[0]
    blane = bl_ref[0]
    onehot_t = (lax.broadcasted_iota(jnp.int32, (GG, RB), 0)
                == blane).astype(jnp.float32)
    sumacc[...] = sumacc[...] + jnp.dot(onehot_t, t,
                                        preferred_element_type=jnp.float32)
    cntacc[...] = cntacc[...] + jnp.sum(onehot_t, axis=1, keepdims=True)

    g_lo = jnp.min(bvec)
    g_hi = jnp.max(bvec)

    def seg(gidx, _):
        m = jnp.max(jnp.where(bvec == gidx, t, -jnp.inf), axis=0,
                    keepdims=True)
        maxacc[gidx] = jnp.maximum(maxacc[gidx], m)
        return 0
    lax.fori_loop(g_lo, g_hi + 1, seg, 0)

    @pl.when(i == NB - 1)
    def _emit():
        meanp = sumacc[...] / jnp.maximum(cntacc[...], 1.0)
        maxp = maxacc[...][:, 0, :]
        out_ref[...] = (
            jnp.dot(maxp, wlmax_ref[...], preferred_element_type=jnp.float32)
            + jnp.dot(meanp, wlmean_ref[...],
                      preferred_element_type=jnp.float32)
            + blin_ref[...])


def _tc_final(accp, hp, degp, b, g, bb, resid, batch_a, batch_b,
              Wl_max, Wl_mean, b_lin):
    row = pl.BlockSpec((RB, DD), lambda i: (i, 0))
    vec = pl.BlockSpec((1, DD), lambda i: (0, 0))
    mat = pl.BlockSpec((DD, DD), lambda i: (0, 0))
    return pl.pallas_call(
        _final_body,
        grid=(NB,),
        in_specs=[
            pl.BlockSpec((NC, RB, DD), lambda i: (0, i, 0)),
            row,
            pl.BlockSpec((NC, RB, DD), lambda i: (0, i, 0)),
            vec, vec, vec,
            row,
            pl.BlockSpec((1, RB, 1), lambda i: (i, 0, 0)),
            pl.BlockSpec((1, 1, RB), lambda i: (i, 0, 0)),
            mat, mat, vec,
        ],
        out_specs=pl.BlockSpec((GG, DD), lambda i: (0, 0)),
        out_shape=jax.ShapeDtypeStruct((GG, DD), jnp.float32),
        scratch_shapes=[
            pltpu.VMEM((GG, 1, DD), jnp.float32),
            pltpu.VMEM((GG, DD), jnp.float32),
            pltpu.VMEM((GG, DD), jnp.float32),
        ],
    )(accp, hp, degp, b, g, bb, resid, batch_a, batch_b,
      Wl_max, Wl_mean, b_lin)


def kernel(x, edge_index, edge_attr, batch,
           W_gcn0, b_gcn0, W_gcn1, b_gcn1, W_gcn2, b_gcn2,
           ln_g0, ln_b0, ln_g1, ln_b1, ln_g2, ln_b2,
           W_lin, b_lin):
    src = edge_index[0].astype(jnp.int32)
    dst = edge_index[1].astype(jnp.int32)
    pad = EP - EE
    srcI = jnp.concatenate(
        [src, jnp.zeros((pad,), jnp.int32)]).reshape(NC, NS, CPT, 1, CHUNK)
    dstI = jnp.concatenate(
        [dst, jnp.full((pad,), NN, jnp.int32)]).reshape(NC, NS, CPT, CHUNK)

    degp = _sc_deg(dstI)
    b0 = b_gcn0.reshape(1, DD)
    b1 = b_gcn1.reshape(1, DD)
    b2 = b_gcn2.reshape(1, DD)
    g0, g1, g2 = (v.reshape(1, DD) for v in (ln_g0, ln_g1, ln_g2))
    bb0, bb1, bb2 = (v.reshape(1, DD) for v in (ln_b0, ln_b1, ln_b2))

    hp0 = _tc_prologue(x, degp, W_gcn0)
    acc0 = _sc_spmm(hp0, srcI, dstI)
    out0, hp1 = _tc_epilogue(acc0, hp0, degp, b0, g0, bb0, W_gcn1, None)
    acc1 = _sc_spmm(hp1, srcI, dstI)
    out1, hp2 = _tc_epilogue(acc1, hp1, degp, b1, g1, bb1, W_gcn2, out0)
    acc2 = _sc_spmm(hp2, srcI, dstI)

    batch_i = batch.astype(jnp.int32)
    batch_a = batch_i.reshape(NB, RB, 1)
    batch_b = batch_i.reshape(NB, 1, RB)
    return _tc_final(acc2, hp2, degp, b2, g2, bb2, out1, batch_a, batch_b,
                     W_lin[:DD], W_lin[DD:], b_lin.reshape(1, DD))
